# baseline (speedup 1.0000x reference)
import jax
import jax.numpy as jnp
from jax import lax
from jax.experimental import pallas as pl
from jax.experimental.pallas import tpu as pltpu
from jax.experimental.pallas import tpu_sc as plsc

N = 10000
D = 128
E = 320000
NP = 10240
NC = 2
NS = 16
NW = NC * NS
B = 128
K = 80
KC = 8
NCH = K // KC
K2 = K + KC
EP = NW * K * B
JUNK = NP - 1
RPT = NP // NS
BLK = 1024


def _mm1_body(x_ref, w_ref, o_ref):
    o_ref[...] = jnp.dot(x_ref[...], w_ref[...],
                         preferred_element_type=jnp.float32)


def _mm1(x_p, w):
    return pl.pallas_call(
        _mm1_body,
        grid=(NP // BLK,),
        in_specs=[pl.BlockSpec((BLK, D), lambda i: (i, 0)),
                  pl.BlockSpec((D, D), lambda i: (0, 0))],
        out_specs=pl.BlockSpec((BLK, D), lambda i: (i, 0)),
        out_shape=jax.ShapeDtypeStruct((NP, D), jnp.float32),
    )(x_p, w)


def _mid_body(y_ref, aA_ref, aB_ref, dA_ref, dB_ref, b1_ref, ws_ref, wn_ref,
              s_ref, z_ref):
    deg = dA_ref[:, 0:1] + dB_ref[:, 0:1]
    h = (y_ref[...] + aA_ref[...] + aB_ref[...]) / (deg + 1.0) + b1_ref[...]
    h = jnp.maximum(h, 0.0)
    s_ref[...] = jnp.dot(h, ws_ref[...], preferred_element_type=jnp.float32)
    z_ref[...] = jnp.dot(h, wn_ref[...], preferred_element_type=jnp.float32)


def _mid(y, aA, aB, dA, dB, b1, ws, wn):
    blk = pl.BlockSpec((BLK, D), lambda i: (i, 0))
    full = pl.BlockSpec((D, D), lambda i: (0, 0))
    return pl.pallas_call(
        _mid_body,
        grid=(NP // BLK,),
        in_specs=[blk, blk, blk, blk, blk,
                  pl.BlockSpec((1, D), lambda i: (0, 0)), full, full],
        out_specs=[blk, blk],
        out_shape=[jax.ShapeDtypeStruct((NP, D), jnp.float32),
                   jax.ShapeDtypeStruct((NP, D), jnp.float32)],
    )(y, aA, aB, dA, dB, b1, ws, wn)


def _fin_body(s_ref, aA_ref, aB_ref, dA_ref, dB_ref, b2_ref, o_ref):
    deg = dA_ref[:, 0:1] + dB_ref[:, 0:1]
    agg = (aA_ref[...] + aB_ref[...]) / jnp.maximum(deg, 1.0)
    o_ref[...] = s_ref[...] + agg + b2_ref[...]


def _fin(s, aA, aB, dA, dB, b2):
    blk = pl.BlockSpec((BLK, D), lambda i: (i, 0))
    return pl.pallas_call(
        _fin_body,
        grid=(NP // BLK,),
        in_specs=[blk, blk, blk, blk, blk,
                  pl.BlockSpec((1, D), lambda i: (0, 0))],
        out_specs=blk,
        out_shape=jax.ShapeDtypeStruct((NP, D), jnp.float32),
    )(s, aA, aB, dA, dB, b2)


def _edge_body(src_hbm, dst_hbm, tab_hbm, zro_hbm, acc_out,
               src_v, dst_v, rows_v, acc_s, sem_g0, sem_g1):
    c = lax.axis_index("c")
    s = lax.axis_index("s")
    wid = s * NC + c
    r0 = s * RPT
    sem_g = (sem_g0, sem_g1)
    pltpu.sync_copy(zro_hbm.at[pl.ds(r0, RPT)], acc_s.at[pl.ds(r0, RPT)])
    plsc.subcore_barrier()

    pltpu.sync_copy(src_hbm.at[wid, pl.ds(0, KC)], src_v)
    pltpu.sync_copy(dst_hbm.at[wid, pl.ds(0, KC)], dst_v)
    pltpu.async_copy(tab_hbm.at[src_v.at[0]], rows_v.at[0], sem_g0)

    def chunk(ch, carry):
        for j in range(KC):
            p = j % 2
            pltpu.make_async_copy(tab_hbm.at[src_v.at[j]], rows_v.at[p],
                                  sem_g[p]).wait()
            if j < KC - 1:
                pltpu.async_copy(tab_hbm.at[src_v.at[j + 1]],
                                 rows_v.at[1 - p], sem_g[1 - p])
                pltpu.sync_copy(rows_v.at[p], acc_s.at[dst_v.at[j]],
                                add=True)
            else:
                pltpu.sync_copy(rows_v.at[p], acc_s.at[dst_v.at[j]],
                                add=True)
                pltpu.sync_copy(src_hbm.at[wid, pl.ds((ch + 1) * KC, KC)],
                                src_v)
                pltpu.sync_copy(dst_hbm.at[wid, pl.ds((ch + 1) * KC, KC)],
                                dst_v)
                pltpu.async_copy(tab_hbm.at[src_v.at[0]], rows_v.at[1 - p],
                                 sem_g[1 - p])
        return carry

    lax.fori_loop(0, NCH, chunk, 0)
    pltpu.make_async_copy(tab_hbm.at[src_v.at[0]], rows_v.at[0],
                          sem_g0).wait()
    plsc.subcore_barrier()
    pltpu.sync_copy(acc_s.at[pl.ds(r0, RPT)], acc_out.at[c, pl.ds(r0, RPT)])


_edge_pass = pl.kernel(
    _edge_body,
    out_type=(jax.ShapeDtypeStruct((NC, NP, D), jnp.float32),),
    mesh=plsc.VectorSubcoreMesh(core_axis_name="c", subcore_axis_name="s"),
    scratch_types=(
        pltpu.VMEM((KC, B), jnp.int32),
        pltpu.VMEM((KC, B), jnp.int32),
        pltpu.VMEM((2, B, D), jnp.float32),
        pltpu.VMEM_SHARED((NP, D), jnp.float32),
        pltpu.SemaphoreType.DMA,
        pltpu.SemaphoreType.DMA,
    ),
)


def _deg_body(dst_hbm, ones_hbm, zro_hbm, deg_out, dst_v, ones_v, dacc_s):
    c = lax.axis_index("c")
    s = lax.axis_index("s")
    wid = s * NC + c
    r0 = s * RPT
    pltpu.sync_copy(zro_hbm.at[pl.ds(r0, RPT)], dacc_s.at[pl.ds(r0, RPT)])
    pltpu.sync_copy(ones_hbm, ones_v)
    plsc.subcore_barrier()

    def chunk(ch, carry):
        pltpu.sync_copy(dst_hbm.at[wid, pl.ds(ch * KC, KC)], dst_v)
        for j in range(KC):
            pltpu.sync_copy(ones_v, dacc_s.at[dst_v.at[j]], add=True)
        return carry

    lax.fori_loop(0, NCH, chunk, 0)
    plsc.subcore_barrier()
    pltpu.sync_copy(dacc_s.at[pl.ds(r0, RPT)], deg_out.at[c, pl.ds(r0, RPT)])


_deg_pass = pl.kernel(
    _deg_body,
    out_type=(jax.ShapeDtypeStruct((NC, NP, D), jnp.float32),),
    mesh=plsc.VectorSubcoreMesh(core_axis_name="c", subcore_axis_name="s"),
    scratch_types=(
        pltpu.VMEM((KC, B), jnp.int32),
        pltpu.VMEM((B, D), jnp.float32),
        pltpu.VMEM_SHARED((NP, D), jnp.float32),
    ),
)


def kernel(x, edge_index, W_neigh1, b1, W_self2, W_neigh2, b2):
    src = edge_index[0]
    dst = edge_index[1]
    x_p = jnp.pad(x, ((0, NP - N), (0, 0)))
    src_p = jnp.concatenate(
        [jnp.pad(src, (0, EP - E)).reshape(NW, K, B),
         jnp.zeros((NW, KC, B), jnp.int32)], axis=1)
    dst_p = jnp.concatenate(
        [jnp.pad(dst, (0, EP - E), constant_values=JUNK).reshape(NW, K, B),
         jnp.full((NW, KC, B), JUNK, jnp.int32)], axis=1)
    zeros = jnp.zeros((NP, D), jnp.float32)
    ones = jnp.ones((B, D), jnp.float32)

    (dg,) = _deg_pass(dst_p, ones, zeros)
    y = _mm1(x_p, W_neigh1)
    (a1,) = _edge_pass(src_p, dst_p, y, zeros)
    s, z = _mid(y, a1[0], a1[1], dg[0], dg[1],
                b1.reshape(1, D), W_self2, W_neigh2)
    (a2,) = _edge_pass(src_p, dst_p, z, zeros)
    out = _fin(s, a2[0], a2[1], dg[0], dg[1], b2.reshape(1, D))
    return out[:N]


# speedup vs baseline: 1.1050x; 1.1050x over previous
import jax
import jax.numpy as jnp
from jax import lax
from jax.experimental import pallas as pl
from jax.experimental.pallas import tpu as pltpu
from jax.experimental.pallas import tpu_sc as plsc

N = 10000
D = 128
E = 320000
NP = 10240
NC = 2
NS = 16
NW = NC * NS
B = 128
K = 80
KC = 8
NCH = K // KC
K0 = 48
K1 = 112
EP = NW * K * B
JUNK = NP - 1
RPT = NP // NS
BLK = 1024


def _mm1_body(x_ref, w_ref, o_ref):
    o_ref[...] = jnp.dot(x_ref[...], w_ref[...],
                         preferred_element_type=jnp.float32)


def _mm1(x_p, w):
    return pl.pallas_call(
        _mm1_body,
        grid=(NP // BLK,),
        in_specs=[pl.BlockSpec((BLK, D), lambda i: (i, 0)),
                  pl.BlockSpec((D, D), lambda i: (0, 0))],
        out_specs=pl.BlockSpec((BLK, D), lambda i: (i, 0)),
        out_shape=jax.ShapeDtypeStruct((NP, D), jnp.float32),
    )(x_p, w)


def _mid_body(y_ref, aA_ref, aB_ref, dA_ref, dB_ref, b1_ref, ws_ref, wn_ref,
              s_ref, z_ref):
    deg = dA_ref[:, 0:1] + dB_ref[:, 0:1]
    h = (y_ref[...] + aA_ref[...] + aB_ref[...]) / (deg + 1.0) + b1_ref[...]
    h = jnp.maximum(h, 0.0)
    s_ref[...] = jnp.dot(h, ws_ref[...], preferred_element_type=jnp.float32)
    z_ref[...] = jnp.dot(h, wn_ref[...], preferred_element_type=jnp.float32)


def _mid(y, aA, aB, dA, dB, b1, ws, wn):
    blk = pl.BlockSpec((BLK, D), lambda i: (i, 0))
    full = pl.BlockSpec((D, D), lambda i: (0, 0))
    return pl.pallas_call(
        _mid_body,
        grid=(NP // BLK,),
        in_specs=[blk, blk, blk, blk, blk,
                  pl.BlockSpec((1, D), lambda i: (0, 0)), full, full],
        out_specs=[blk, blk],
        out_shape=[jax.ShapeDtypeStruct((NP, D), jnp.float32),
                   jax.ShapeDtypeStruct((NP, D), jnp.float32)],
    )(y, aA, aB, dA, dB, b1, ws, wn)


def _fin_body(s_ref, aA_ref, aB_ref, dA_ref, dB_ref, b2_ref, o_ref):
    deg = dA_ref[:, 0:1] + dB_ref[:, 0:1]
    agg = (aA_ref[...] + aB_ref[...]) / jnp.maximum(deg, 1.0)
    o_ref[...] = s_ref[...] + agg + b2_ref[...]


def _fin(s, aA, aB, dA, dB, b2):
    blk = pl.BlockSpec((BLK, D), lambda i: (i, 0))
    return pl.pallas_call(
        _fin_body,
        grid=(NP // BLK,),
        in_specs=[blk, blk, blk, blk, blk,
                  pl.BlockSpec((1, D), lambda i: (0, 0))],
        out_specs=blk,
        out_shape=jax.ShapeDtypeStruct((NP, D), jnp.float32),
    )(s, aA, aB, dA, dB, b2)


def _edge_body(src_hbm, dst_hbm, tab_hbm, zro_hbm, acc_out,
               src_v, dst_v, rows_v, acc_s, sem):
    c = lax.axis_index("c")
    s = lax.axis_index("s")
    wid = s * NC + c
    r0 = s * RPT
    ncz = lax.select(c == 0, K0 // KC, K1 // KC)
    pltpu.sync_copy(zro_hbm.at[pl.ds(r0, RPT)], acc_s.at[pl.ds(r0, RPT)])
    plsc.subcore_barrier()

    def chunk(ch, carry):
        pltpu.sync_copy(src_hbm.at[wid, pl.ds(ch * KC, KC)], src_v)
        pltpu.sync_copy(dst_hbm.at[wid, pl.ds(ch * KC, KC)], dst_v)
        for j in range(KC):
            pltpu.async_copy(tab_hbm.at[src_v.at[j]], rows_v, sem).wait()
            pltpu.sync_copy(rows_v, acc_s.at[dst_v.at[j]], add=True)
        return carry

    lax.fori_loop(0, ncz, chunk, 0)
    plsc.subcore_barrier()
    pltpu.sync_copy(acc_s.at[pl.ds(r0, RPT)], acc_out.at[c, pl.ds(r0, RPT)])


_edge_pass = pl.kernel(
    _edge_body,
    out_type=(jax.ShapeDtypeStruct((NC, NP, D), jnp.float32),),
    mesh=plsc.VectorSubcoreMesh(core_axis_name="c", subcore_axis_name="s"),
    scratch_types=(
        pltpu.VMEM((KC, B), jnp.int32),
        pltpu.VMEM((KC, B), jnp.int32),
        pltpu.VMEM((B, D), jnp.float32),
        pltpu.VMEM_SHARED((NP, D), jnp.float32),
        pltpu.SemaphoreType.DMA,
    ),
)


def _deg_body(dst_hbm, ones_hbm, zro_hbm, deg_out, dst_v, ones_v, dacc_s):
    c = lax.axis_index("c")
    s = lax.axis_index("s")
    wid = s * NC + c
    r0 = s * RPT
    pltpu.sync_copy(zro_hbm.at[pl.ds(r0, RPT)], dacc_s.at[pl.ds(r0, RPT)])
    pltpu.sync_copy(ones_hbm, ones_v)
    plsc.subcore_barrier()

    def chunk(ch, carry):
        pltpu.sync_copy(dst_hbm.at[wid, pl.ds(ch * KC, KC)], dst_v)
        for j in range(KC):
            pltpu.sync_copy(ones_v, dacc_s.at[dst_v.at[j]], add=True)
        return carry

    lax.fori_loop(0, NCH, chunk, 0)
    plsc.subcore_barrier()
    pltpu.sync_copy(dacc_s.at[pl.ds(r0, RPT)], deg_out.at[c, pl.ds(r0, RPT)])


_deg_pass = pl.kernel(
    _deg_body,
    out_type=(jax.ShapeDtypeStruct((NC, NP, D), jnp.float32),),
    mesh=plsc.VectorSubcoreMesh(core_axis_name="c", subcore_axis_name="s"),
    scratch_types=(
        pltpu.VMEM((KC, B), jnp.int32),
        pltpu.VMEM((B, D), jnp.float32),
        pltpu.VMEM_SHARED((NP, D), jnp.float32),
    ),
)


def kernel(x, edge_index, W_neigh1, b1, W_self2, W_neigh2, b2):
    src = edge_index[0]
    dst = edge_index[1]
    x_p = jnp.pad(x, ((0, NP - N), (0, 0)))

    def asym_layout(idx, fill):
        flat = jnp.pad(idx, (0, EP - E), constant_values=fill)
        e0 = flat[:NS * K0 * B].reshape(NS, 1, K0, B)
        e0 = jnp.pad(e0, ((0, 0), (0, 0), (0, K1 - K0), (0, 0)),
                     constant_values=fill)
        e1 = flat[NS * K0 * B:].reshape(NS, 1, K1, B)
        return jnp.concatenate([e0, e1], axis=1).reshape(NW, K1, B)

    src_p = asym_layout(src, 0)
    dst_p = asym_layout(dst, JUNK)
    dst_d = jnp.pad(dst, (0, EP - E), constant_values=JUNK).reshape(NW, K, B)
    zeros = jnp.zeros((NP, D), jnp.float32)
    ones = jnp.ones((B, D), jnp.float32)

    (dg,) = _deg_pass(dst_d, ones, zeros)
    y = _mm1(x_p, W_neigh1)
    (a1,) = _edge_pass(src_p, dst_p, y, zeros)
    s, z = _mid(y, a1[0], a1[1], dg[0], dg[1],
                b1.reshape(1, D), W_self2, W_neigh2)
    (a2,) = _edge_pass(src_p, dst_p, z, zeros)
    out = _fin(s, a2[0], a2[1], dg[0], dg[1], b2.reshape(1, D))
    return out[:N]


# speedup vs baseline: 1.2365x; 1.1190x over previous
import jax
import jax.numpy as jnp
from jax import lax
from jax.experimental import pallas as pl
from jax.experimental.pallas import tpu as pltpu
from jax.experimental.pallas import tpu_sc as plsc

N = 10000
D = 128
E = 320000
NP = 10240
NC = 2
NS = 16
NW = NC * NS
B = 128
K = 80
KC = 8
NCH = K // KC
K0 = 112
K1 = 48
EP = NW * K * B
JUNK = NP - 1
RPT = NP // NS
BLK = 1024


def _mm1_body(x_ref, w_ref, o_ref):
    o_ref[...] = jnp.dot(x_ref[...], w_ref[...],
                         preferred_element_type=jnp.float32)


def _mm1(x_p, w):
    return pl.pallas_call(
        _mm1_body,
        grid=(NP // BLK,),
        in_specs=[pl.BlockSpec((BLK, D), lambda i: (i, 0)),
                  pl.BlockSpec((D, D), lambda i: (0, 0))],
        out_specs=pl.BlockSpec((BLK, D), lambda i: (i, 0)),
        out_shape=jax.ShapeDtypeStruct((NP, D), jnp.float32),
    )(x_p, w)


def _mid_body(y_ref, aA_ref, aB_ref, dA_ref, dB_ref, b1_ref, ws_ref, wn_ref,
              s_ref, z_ref):
    deg = dA_ref[:, 0:1] + dB_ref[:, 0:1]
    h = (y_ref[...] + aA_ref[...] + aB_ref[...]) / (deg + 1.0) + b1_ref[...]
    h = jnp.maximum(h, 0.0)
    s_ref[...] = jnp.dot(h, ws_ref[...], preferred_element_type=jnp.float32)
    z_ref[...] = jnp.dot(h, wn_ref[...], preferred_element_type=jnp.float32)


def _mid(y, aA, aB, dA, dB, b1, ws, wn):
    blk = pl.BlockSpec((BLK, D), lambda i: (i, 0))
    full = pl.BlockSpec((D, D), lambda i: (0, 0))
    return pl.pallas_call(
        _mid_body,
        grid=(NP // BLK,),
        in_specs=[blk, blk, blk, blk, blk,
                  pl.BlockSpec((1, D), lambda i: (0, 0)), full, full],
        out_specs=[blk, blk],
        out_shape=[jax.ShapeDtypeStruct((NP, D), jnp.float32),
                   jax.ShapeDtypeStruct((NP, D), jnp.float32)],
    )(y, aA, aB, dA, dB, b1, ws, wn)


def _fin_body(s_ref, aA_ref, aB_ref, dA_ref, dB_ref, b2_ref, o_ref):
    deg = dA_ref[:, 0:1] + dB_ref[:, 0:1]
    agg = (aA_ref[...] + aB_ref[...]) / jnp.maximum(deg, 1.0)
    o_ref[...] = s_ref[...] + agg + b2_ref[...]


def _fin(s, aA, aB, dA, dB, b2):
    blk = pl.BlockSpec((BLK, D), lambda i: (i, 0))
    return pl.pallas_call(
        _fin_body,
        grid=(NP // BLK,),
        in_specs=[blk, blk, blk, blk, blk,
                  pl.BlockSpec((1, D), lambda i: (0, 0))],
        out_specs=blk,
        out_shape=jax.ShapeDtypeStruct((NP, D), jnp.float32),
    )(s, aA, aB, dA, dB, b2)


def _edge_body(src_hbm, dst_hbm, tab_hbm, zro_hbm, acc_out,
               src_v, dst_v, rows_v, acc_s, sem):
    c = lax.axis_index("c")
    s = lax.axis_index("s")
    wid = s * NC + c
    r0 = s * RPT
    ncz = lax.select(c == 0, K0 // KC, K1 // KC)
    pltpu.sync_copy(zro_hbm.at[pl.ds(r0, RPT)], acc_s.at[pl.ds(r0, RPT)])
    plsc.subcore_barrier()

    def chunk(ch, carry):
        pltpu.sync_copy(src_hbm.at[wid, pl.ds(ch * KC, KC)], src_v)
        pltpu.sync_copy(dst_hbm.at[wid, pl.ds(ch * KC, KC)], dst_v)
        for j in range(KC):
            pltpu.async_copy(tab_hbm.at[src_v.at[j]], rows_v, sem).wait()
            pltpu.sync_copy(rows_v, acc_s.at[dst_v.at[j]], add=True)
        return carry

    lax.fori_loop(0, ncz, chunk, 0)
    plsc.subcore_barrier()
    pltpu.sync_copy(acc_s.at[pl.ds(r0, RPT)], acc_out.at[c, pl.ds(r0, RPT)])


_edge_pass = pl.kernel(
    _edge_body,
    out_type=(jax.ShapeDtypeStruct((NC, NP, D), jnp.float32),),
    mesh=plsc.VectorSubcoreMesh(core_axis_name="c", subcore_axis_name="s"),
    scratch_types=(
        pltpu.VMEM((KC, B), jnp.int32),
        pltpu.VMEM((KC, B), jnp.int32),
        pltpu.VMEM((B, D), jnp.float32),
        pltpu.VMEM_SHARED((NP, D), jnp.float32),
        pltpu.SemaphoreType.DMA,
    ),
)


def _deg_body(dst_hbm, ones_hbm, zro_hbm, deg_out, dst_v, ones_v, dacc_s):
    c = lax.axis_index("c")
    s = lax.axis_index("s")
    wid = s * NC + c
    r0 = s * RPT
    pltpu.sync_copy(zro_hbm.at[pl.ds(r0, RPT)], dacc_s.at[pl.ds(r0, RPT)])
    pltpu.sync_copy(ones_hbm, ones_v)
    plsc.subcore_barrier()

    def chunk(ch, carry):
        pltpu.sync_copy(dst_hbm.at[wid, pl.ds(ch * KC, KC)], dst_v)
        for j in range(KC):
            pltpu.sync_copy(ones_v, dacc_s.at[dst_v.at[j]], add=True)
        return carry

    lax.fori_loop(0, NCH, chunk, 0)
    plsc.subcore_barrier()
    pltpu.sync_copy(dacc_s.at[pl.ds(r0, RPT)], deg_out.at[c, pl.ds(r0, RPT)])


_deg_pass = pl.kernel(
    _deg_body,
    out_type=(jax.ShapeDtypeStruct((NC, NP, D), jnp.float32),),
    mesh=plsc.VectorSubcoreMesh(core_axis_name="c", subcore_axis_name="s"),
    scratch_types=(
        pltpu.VMEM((KC, B), jnp.int32),
        pltpu.VMEM((B, D), jnp.float32),
        pltpu.VMEM_SHARED((NP, D), jnp.float32),
    ),
)


def kernel(x, edge_index, W_neigh1, b1, W_self2, W_neigh2, b2):
    src = edge_index[0]
    dst = edge_index[1]
    x_p = jnp.pad(x, ((0, NP - N), (0, 0)))

    kmax = max(K0, K1)

    def asym_layout(idx, fill):
        flat = jnp.pad(idx, (0, EP - E), constant_values=fill)
        e0 = flat[:NS * K0 * B].reshape(NS, 1, K0, B)
        e0 = jnp.pad(e0, ((0, 0), (0, 0), (0, kmax - K0), (0, 0)),
                     constant_values=fill)
        e1 = flat[NS * K0 * B:].reshape(NS, 1, K1, B)
        e1 = jnp.pad(e1, ((0, 0), (0, 0), (0, kmax - K1), (0, 0)),
                     constant_values=fill)
        return jnp.concatenate([e0, e1], axis=1).reshape(NW, kmax, B)

    src_p = asym_layout(src, 0)
    dst_p = asym_layout(dst, JUNK)
    dst_d = jnp.pad(dst, (0, EP - E), constant_values=JUNK).reshape(NW, K, B)
    zeros = jnp.zeros((NP, D), jnp.float32)
    ones = jnp.ones((B, D), jnp.float32)

    (dg,) = _deg_pass(dst_d, ones, zeros)
    y = _mm1(x_p, W_neigh1)
    (a1,) = _edge_pass(src_p, dst_p, y, zeros)
    s, z = _mid(y, a1[0], a1[1], dg[0], dg[1],
                b1.reshape(1, D), W_self2, W_neigh2)
    (a2,) = _edge_pass(src_p, dst_p, z, zeros)
    out = _fin(s, a2[0], a2[1], dg[0], dg[1], b2.reshape(1, D))
    return out[:N]
